# baseline (speedup 1.0000x reference)
import functools

import jax
import jax.numpy as jnp
from jax import lax
from jax.experimental import pallas as pl
from jax.experimental.pallas import tpu as pltpu
from jax.experimental.pallas import tpu_sc as plsc

_P = 0.0005
_FMIN = float(jnp.finfo(jnp.float32).min)

_CHUNK = 2048
_NCHUNK = 6
_WCOLS = _CHUNK * _NCHUNK
_SC_COLS = _WCOLS * 8


def _desc_rank_row(w):
    r, n = w.shape
    wd = w[:, :, None]
    we = w[:, None, :]
    d_idx = jax.lax.broadcasted_iota(jnp.int32, (r, n, n), 1)
    e_idx = jax.lax.broadcasted_iota(jnp.int32, (r, n, n), 2)
    beats = (we > wd) | ((we == wd) & (e_idx < d_idx))
    return jnp.sum(beats.astype(jnp.int32), axis=2)


def _desc_rank_col(w):
    n, b = w.shape
    wd = w[:, None, :]
    we = w[None, :, :]
    d_idx = jax.lax.broadcasted_iota(jnp.int32, (n, n, b), 0)
    e_idx = jax.lax.broadcasted_iota(jnp.int32, (n, n, b), 1)
    beats = (we > wd) | ((we == wd) & (e_idx < d_idx))
    return jnp.sum(beats.astype(jnp.int32), axis=1)


def _sc_stats_body(x_ref, xf_ref, m_out, s_out, buf, resm, ress):
    nc = 2
    wid = lax.axis_index("s") * nc + lax.axis_index("c")
    arr = wid // 16
    g = (wid // 8) % 2
    q = wid % 8

    nvec = _CHUNK // 16

    def run(src_ref):
        def chunk_body(t, carry):
            pltpu.sync_copy(
                src_ref.at[pl.ds(8 * g, 8),
                           pl.ds(q * _WCOLS + t * _CHUNK, _CHUNK)],
                buf)
            ms = list(carry)
            for r in range(8):
                def vec_body(i, c2, r=r):
                    acc = list(c2)
                    for u in range(4):
                        v = buf[r, pl.ds((4 * i + u) * 16, 16)]
                        m2, s2 = acc[2 * u], acc[2 * u + 1]
                        mn = jnp.maximum(m2, v)
                        acc[2 * u] = mn
                        acc[2 * u + 1] = (s2 * jnp.exp(m2 - mn)
                                          + jnp.exp(v - mn))
                    return tuple(acc)

                ms[r] = lax.fori_loop(0, nvec // 4, vec_body, ms[r])
            return tuple(ms)

        m0 = jnp.full((16,), _FMIN, jnp.float32)
        s0 = jnp.zeros((16,), jnp.float32)
        carry = lax.fori_loop(0, _NCHUNK, chunk_body,
                              ((m0, s0) * 4,) * 8)

        for r in range(8):
            acc = carry[r]
            m = acc[0]
            for u in range(1, 4):
                m = jnp.maximum(m, acc[2 * u])
            s = jnp.zeros((16,), jnp.float32)
            for u in range(4):
                s = s + acc[2 * u + 1] * jnp.exp(acc[2 * u] - m)
            resm[...] = m
            ress[...] = s
            off = arr * 2048 + g * 1024 + r * 128 + q * 16
            pltpu.sync_copy(resm, m_out.at[pl.ds(off, 16)])
            pltpu.sync_copy(ress, s_out.at[pl.ds(off, 16)])

    @pl.when(arr == 0)
    def _a0():
        run(x_ref)

    @pl.when(arr == 1)
    def _a1():
        run(xf_ref)


def _mm_body(feat_ref, wt_ref, b_ref, out_ref):
    y = jax.lax.dot_general(feat_ref[...], wt_ref[...],
                            (((1,), (0,)), ((), ())),
                            preferred_element_type=jnp.float32)
    out_ref[...] = y + b_ref[...][None, :]


def _fix_body(feat_ref, wt_ref, b_ref, x_ref, xf_ref, xt_ref, xft_ref,
              mst_ref, sst_ref, prev_ref, out_ref, *, k, b_sz, c, tail_blk):
    tail_w = c - _SC_COLS
    tvalid = (jax.lax.broadcasted_iota(jnp.int32, (b_sz, tail_blk), 1)
              < tail_w)

    def _finish(a, xt, x0):
        xr = jnp.where(tvalid, xt, _FMIN).reshape(b_sz, tail_blk // 128, 128)
        mt = jnp.max(xr, axis=1)
        st = jnp.sum(jnp.exp(xr - mt[:, None, :]), axis=1)
        m2 = mst_ref[a]
        s2 = sst_ref[a]
        m_row = jnp.maximum(jnp.max(m2, axis=1, keepdims=True),
                            jnp.max(mt, axis=1, keepdims=True))
        s_row = (jnp.sum(s2 * jnp.exp(m2 - m_row), axis=1, keepdims=True)
                 + jnp.sum(st * jnp.exp(mt - m_row), axis=1, keepdims=True))
        return jnp.exp(x0 - m_row) / s_row

    h = _finish(0, xt_ref[...], x_ref[:, 0:1])
    hf = _finish(1, xft_ref[...], xf_ref[:, 0:1])

    eye = (jax.lax.broadcasted_iota(jnp.int32, (b_sz, b_sz), 0)
           == jax.lax.broadcasted_iota(jnp.int32, (b_sz, b_sz), 1))

    def _trow(col):
        return jnp.sum(jnp.where(eye, col, 0), axis=0, keepdims=True)

    def _tcol(row):
        return jnp.sum(jnp.where(eye, row, 0), axis=1, keepdims=True)

    rank_h = _desc_rank_row(_trow(h))
    rank_hf = _desc_rank_row(_trow(hf))
    pair = rank_hf == _tcol(rank_h)

    wt16 = wt_ref[:, 0:b_sz]
    rd = _desc_rank_col(wt16)
    w_src = jnp.sum(jnp.where(pair[None, :, :], wt16[:, None, :], 0.0),
                    axis=2)
    r_src = jnp.sum(jnp.where(pair[None, :, :], rd[:, None, :], 0),
                    axis=2)
    take = r_src[None, :, :] == rd[:, None, :]
    newval = jnp.sum(jnp.where(take, w_src[None, :, :], 0.0), axis=1)
    wt16_mod = jnp.where(rd < k, newval, wt16)

    b16 = b_ref[0:b_sz][None, :]
    b16_mod = _trow(jnp.sum(jnp.where(pair, b16, 0.0),
                            axis=1, keepdims=True))

    y16 = jax.lax.dot_general(feat_ref[...], wt16_mod,
                              (((1,), (0,)), ((), ())),
                              preferred_element_type=jnp.float32)
    out_ref[...] = prev_ref[...]
    out_ref[:, 0:b_sz] = y16 + b16_mod


def kernel(features, features_f, output, output_f, weight_matrix, bias):
    del features_f
    b_sz, d = features.shape
    c = weight_matrix.shape[0]
    k = int(round(c * _P))
    blk = 16384
    n_blocks = pl.cdiv(c, blk)
    wt = weight_matrix.T

    mesh = plsc.VectorSubcoreMesh(core_axis_name="c", subcore_axis_name="s")
    sc_stats = pl.kernel(
        _sc_stats_body,
        mesh=mesh,
        out_type=[jax.ShapeDtypeStruct((4096,), jnp.float32),
                  jax.ShapeDtypeStruct((4096,), jnp.float32)],
        scratch_types=[pltpu.VMEM((8, _CHUNK), jnp.float32),
                       pltpu.VMEM((16,), jnp.float32),
                       pltpu.VMEM((16,), jnp.float32)],
    )
    m_flat, s_flat = sc_stats(output, output_f)
    m_st = m_flat.reshape(2, b_sz, 128)
    s_st = s_flat.reshape(2, b_sz, 128)

    out_main = pl.pallas_call(
        _mm_body,
        grid=(n_blocks,),
        in_specs=[
            pl.BlockSpec((b_sz, d), lambda i: (0, 0)),
            pl.BlockSpec((d, blk), lambda i: (0, i)),
            pl.BlockSpec((blk,), lambda i: (i,)),
        ],
        out_specs=pl.BlockSpec((b_sz, blk), lambda i: (0, i)),
        out_shape=jax.ShapeDtypeStruct((b_sz, c), jnp.float32),
        compiler_params=pltpu.CompilerParams(
            dimension_semantics=("parallel",)),
    )(features, wt, bias)

    tail_blk = 2048
    tail_idx = _SC_COLS // tail_blk
    fix_body = functools.partial(_fix_body, k=k, b_sz=b_sz, c=c,
                                 tail_blk=tail_blk)
    head2 = lambda shape: pl.BlockSpec(shape, lambda i: (0, 0))
    st_spec = pl.BlockSpec((2, b_sz, 128), lambda i: (0, 0, 0))
    tail_spec = pl.BlockSpec((b_sz, tail_blk), lambda i: (0, tail_idx))
    return pl.pallas_call(
        fix_body,
        grid=(1,),
        in_specs=[
            head2((b_sz, d)),
            head2((d, 128)),
            pl.BlockSpec((128,), lambda i: (0,)),
            head2((b_sz, 128)),
            head2((b_sz, 128)),
            tail_spec,
            tail_spec,
            st_spec, st_spec,
            head2((b_sz, 128)),
        ],
        out_specs=head2((b_sz, 128)),
        out_shape=jax.ShapeDtypeStruct((b_sz, c), jnp.float32),
        input_output_aliases={9: 0},
    )(features, wt, bias, output, output_f, output, output_f,
      m_st, s_st, out_main)


# speedup vs baseline: 1.8185x; 1.8185x over previous
import functools

import jax
import jax.numpy as jnp
from jax.experimental import pallas as pl
from jax.experimental.pallas import tpu as pltpu

_P = 0.0005
_FMIN = float(jnp.finfo(jnp.float32).min)


def _desc_rank_row(w):
    r, n = w.shape
    wd = w[:, :, None]
    we = w[:, None, :]
    d_idx = jax.lax.broadcasted_iota(jnp.int32, (r, n, n), 1)
    e_idx = jax.lax.broadcasted_iota(jnp.int32, (r, n, n), 2)
    beats = (we > wd) | ((we == wd) & (e_idx < d_idx))
    return jnp.sum(beats.astype(jnp.int32), axis=2)


def _desc_rank_col(w):
    n, b = w.shape
    wd = w[:, None, :]
    we = w[None, :, :]
    d_idx = jax.lax.broadcasted_iota(jnp.int32, (n, n, b), 0)
    e_idx = jax.lax.broadcasted_iota(jnp.int32, (n, n, b), 1)
    beats = (we > wd) | ((we == wd) & (e_idx < d_idx))
    return jnp.sum(beats.astype(jnp.int32), axis=1)


def _fused_body(feat_ref, x_ref, xf_ref, wt_ref, b_ref, out_ref,
                m_ref, s_ref, mf_ref, sf_ref, *, n_blocks, blk, c, k, b_sz):
    i = pl.program_id(0)
    j = (i + 1) % n_blocks

    @pl.when(i == 0)
    def _init():
        neg = jnp.full((b_sz, blk), _FMIN, jnp.float32)
        zero = jnp.zeros((b_sz, blk), jnp.float32)
        m_ref[...] = neg
        s_ref[...] = zero
        mf_ref[...] = neg
        sf_ref[...] = zero

    def _acc(x, m_r, s_r):
        m_old = m_r[...]
        m_new = jnp.maximum(m_old, x)
        s_r[...] = s_r[...] * jnp.exp(m_old - m_new) + jnp.exp(x - m_new)
        m_r[...] = m_new

    rem_w = c - (n_blocks - 1) * blk
    if rem_w == blk:
        _acc(x_ref[...], m_ref, s_ref)
        _acc(xf_ref[...], mf_ref, sf_ref)
    else:
        @pl.when(j != n_blocks - 1)
        def _full():
            _acc(x_ref[...], m_ref, s_ref)
            _acc(xf_ref[...], mf_ref, sf_ref)

        @pl.when(j == n_blocks - 1)
        def _ragged():
            valid = (jax.lax.broadcasted_iota(jnp.int32, (b_sz, blk), 1)
                     < rem_w)
            _acc(jnp.where(valid, x_ref[...], _FMIN), m_ref, s_ref)
            _acc(jnp.where(valid, xf_ref[...], _FMIN), mf_ref, sf_ref)

    feat = feat_ref[...]
    dims = (((1,), (0,)), ((), ()))
    y = jax.lax.dot_general(feat, wt_ref[...], dims,
                            preferred_element_type=jnp.float32)
    out_ref[...] = y + b_ref[...][None, :]

    @pl.when(i == n_blocks - 1)
    def _last():
        def _finish(m_r, s_r, x0):
            m_vec = m_r[...]
            m_row = jnp.max(m_vec, axis=1, keepdims=True)
            s_row = jnp.sum(s_r[...] * jnp.exp(m_vec - m_row),
                            axis=1, keepdims=True)
            return jnp.exp(x0 - m_row) / s_row

        h = _finish(m_ref, s_ref, x_ref[:, 0:1])
        hf = _finish(mf_ref, sf_ref, xf_ref[:, 0:1])

        eye = (jax.lax.broadcasted_iota(jnp.int32, (b_sz, b_sz), 0)
               == jax.lax.broadcasted_iota(jnp.int32, (b_sz, b_sz), 1))

        def _trow(col):
            return jnp.sum(jnp.where(eye, col, 0), axis=0, keepdims=True)

        def _tcol(row):
            return jnp.sum(jnp.where(eye, row, 0), axis=1, keepdims=True)

        rank_h = _desc_rank_row(_trow(h))
        rank_hf = _desc_rank_row(_trow(hf))
        pair = rank_hf == _tcol(rank_h)

        wt16 = wt_ref[:, 0:b_sz]
        rd = _desc_rank_col(wt16)
        w_src = jnp.sum(jnp.where(pair[None, :, :], wt16[:, None, :], 0.0),
                        axis=2)
        r_src = jnp.sum(jnp.where(pair[None, :, :], rd[:, None, :], 0),
                        axis=2)
        take = r_src[None, :, :] == rd[:, None, :]
        newval = jnp.sum(jnp.where(take, w_src[None, :, :], 0.0), axis=1)
        wt16_mod = jnp.where(rd < k, newval, wt16)

        b16 = b_ref[0:b_sz][None, :]
        b16_mod = _trow(jnp.sum(jnp.where(pair, b16, 0.0),
                                axis=1, keepdims=True))

        y16 = jax.lax.dot_general(feat, wt16_mod, dims,
                                  preferred_element_type=jnp.float32)
        out_ref[:, 0:b_sz] = y16 + b16_mod


def kernel(features, features_f, output, output_f, weight_matrix, bias):
    del features_f
    b_sz, d = features.shape
    c = weight_matrix.shape[0]
    k = int(round(c * _P))
    blk = 16384
    n_blocks = pl.cdiv(c, blk)
    wt = weight_matrix.T

    shift = lambda i: (i + 1) % n_blocks
    body = functools.partial(_fused_body, n_blocks=n_blocks, blk=blk, c=c,
                             k=k, b_sz=b_sz)
    return pl.pallas_call(
        body,
        grid=(n_blocks,),
        in_specs=[
            pl.BlockSpec((b_sz, d), lambda i: (0, 0)),
            pl.BlockSpec((b_sz, blk), lambda i: (0, shift(i))),
            pl.BlockSpec((b_sz, blk), lambda i: (0, shift(i))),
            pl.BlockSpec((d, blk), lambda i: (0, shift(i))),
            pl.BlockSpec((blk,), lambda i: (shift(i),)),
        ],
        out_specs=pl.BlockSpec((b_sz, blk), lambda i: (0, shift(i))),
        out_shape=jax.ShapeDtypeStruct((b_sz, c), jnp.float32),
        scratch_shapes=[pltpu.VMEM((b_sz, blk), jnp.float32)] * 4,
        compiler_params=pltpu.CompilerParams(
            dimension_semantics=("arbitrary",)),
    )(features, output, output_f, wt, bias)
